# Initial kernel scaffold; baseline (speedup 1.0000x reference)
import jax
import jax.numpy as jnp
from jax.experimental import pallas as pl


def kernel(inputs, W):
    raise NotImplementedError("write your pallas kernel here")


# speedup vs baseline: 3.5115x; 3.5115x over previous
import functools

import jax
import jax.numpy as jnp
from jax.experimental import pallas as pl
from jax.experimental.pallas import tpu as pltpu

_K = 1024
_D = 64
_BLK = 2048
_COMMIT = 0.25


def _vq_body(x_ref, w_ref, enc_ref, q_ref, loss_ref, ppl_ref,
             sse_acc, cnt_acc, *, n_tokens, n_blocks):
    i = pl.program_id(0)
    x = x_ref[...]
    w = w_ref[...]
    m = jax.lax.dot_general(x, w, (((1,), (1,)), ((), ())),
                            preferred_element_type=jnp.float32)
    xsq = jnp.sum(x * x, axis=1, keepdims=True)
    esq = jnp.sum(w * w, axis=1)[None, :]
    d = (xsq + esq) - 2.0 * m
    dmin = jnp.min(d, axis=1, keepdims=True)
    col = jax.lax.broadcasted_iota(jnp.int32, d.shape, 1)
    idx = jnp.min(jnp.where(d == dmin, col, _K), axis=1, keepdims=True)
    one_hot = (col == idx).astype(jnp.float32)
    enc_ref[...] = one_hot
    q = jnp.dot(one_hot, w, preferred_element_type=jnp.float32)
    q_ref[...] = x + (q - x)
    sse = jnp.sum((q - x) ** 2)
    cnt = jnp.sum(one_hot, axis=0, keepdims=True)

    @pl.when(i == 0)
    def _init():
        sse_acc[0, 0] = sse
        cnt_acc[...] = cnt

    @pl.when(i > 0)
    def _accum():
        sse_acc[0, 0] += sse
        cnt_acc[...] += cnt

    @pl.when(i == n_blocks - 1)
    def _finalize():
        mse = sse_acc[0, 0] / (n_tokens * _D)
        loss_ref[...] = jnp.full((1, 1), (1.0 + _COMMIT) * mse, jnp.float32)
        avg = cnt_acc[...] * (1.0 / n_tokens)
        ent = -jnp.sum(avg * jnp.log(avg + 1e-10), keepdims=True)
        ppl_ref[...] = jnp.exp(ent)


def kernel(inputs, W):
    input_shape = inputs.shape
    flat = inputs.reshape(-1, _D)
    n = flat.shape[0]
    n_blocks = n // _BLK
    body = functools.partial(_vq_body, n_tokens=n, n_blocks=n_blocks)
    enc, q, loss, ppl = pl.pallas_call(
        body,
        grid=(n_blocks,),
        in_specs=[
            pl.BlockSpec((_BLK, _D), lambda i: (i, 0)),
            pl.BlockSpec((_K, _D), lambda i: (0, 0)),
        ],
        out_specs=[
            pl.BlockSpec((_BLK, _K), lambda i: (i, 0)),
            pl.BlockSpec((_BLK, _D), lambda i: (i, 0)),
            pl.BlockSpec((1, 1), lambda i: (0, 0)),
            pl.BlockSpec((1, 1), lambda i: (0, 0)),
        ],
        out_shape=[
            jax.ShapeDtypeStruct((n, _K), jnp.float32),
            jax.ShapeDtypeStruct((n, _D), jnp.float32),
            jax.ShapeDtypeStruct((1, 1), jnp.float32),
            jax.ShapeDtypeStruct((1, 1), jnp.float32),
        ],
        scratch_shapes=[
            pltpu.SMEM((1, 1), jnp.float32),
            pltpu.VMEM((1, _K), jnp.float32),
        ],
    )(flat, W)
    return (loss[0, 0], q.reshape(input_shape), ppl[0, 0], enc)


# speedup vs baseline: 3.6663x; 1.0441x over previous
import functools

import jax
import jax.numpy as jnp
from jax.experimental import pallas as pl
from jax.experimental.pallas import tpu as pltpu

_K = 1024
_D = 64
_BLK = 2048
_COMMIT = 0.25


def _vq_body(x_ref, w_ref, enc_ref, q_ref, loss_ref, ppl_ref,
             sse_acc, cnt_acc, cnt_blk, *, n_tokens, n_blocks):
    i = pl.program_id(0)
    x = x_ref[...]
    w = w_ref[...]
    m2 = jax.lax.dot_general(x * (-2.0), w, (((1,), (1,)), ((), ())),
                             preferred_element_type=jnp.float32)
    xsq = jnp.sum(x * x, axis=1, keepdims=True)
    esq = jnp.sum(w * w, axis=1)[None, :]
    d = (xsq + esq) + m2
    dmin = jnp.min(d, axis=1, keepdims=True)
    mask = (d == dmin).astype(jnp.float32)
    cnt = jnp.sum(mask, axis=0, keepdims=True)
    total = jnp.sum(cnt)
    enc_ref[...] = mask
    q = jnp.dot(mask, w, preferred_element_type=jnp.float32)
    q_ref[...] = x + (q - x)
    cnt_blk[...] = cnt

    @pl.when(total != _BLK)
    def _fix_ties():
        col = jax.lax.broadcasted_iota(jnp.int32, d.shape, 1)
        idx = jnp.min(jnp.where(d == dmin, col, _K), axis=1, keepdims=True)
        one_hot = (col == idx).astype(jnp.float32)
        enc_ref[...] = one_hot
        qf = jnp.dot(one_hot, w, preferred_element_type=jnp.float32)
        q_ref[...] = x + (qf - x)
        cnt_blk[...] = jnp.sum(one_hot, axis=0, keepdims=True)

    sse = jnp.sum(dmin)

    @pl.when(i == 0)
    def _init():
        sse_acc[0, 0] = sse
        cnt_acc[...] = cnt_blk[...]

    @pl.when(i > 0)
    def _accum():
        sse_acc[0, 0] += sse
        cnt_acc[...] += cnt_blk[...]

    @pl.when(i == n_blocks - 1)
    def _finalize():
        mse = sse_acc[0, 0] / (n_tokens * _D)
        loss_ref[...] = jnp.full((1, 1), (1.0 + _COMMIT) * mse, jnp.float32)
        avg = cnt_acc[...] * (1.0 / n_tokens)
        ent = -jnp.sum(avg * jnp.log(avg + 1e-10), keepdims=True)
        ppl_ref[...] = jnp.exp(ent)


def kernel(inputs, W):
    input_shape = inputs.shape
    flat = inputs.reshape(-1, _D)
    n = flat.shape[0]
    n_blocks = n // _BLK
    body = functools.partial(_vq_body, n_tokens=n, n_blocks=n_blocks)
    enc, q, loss, ppl = pl.pallas_call(
        body,
        grid=(n_blocks,),
        in_specs=[
            pl.BlockSpec((_BLK, _D), lambda i: (i, 0)),
            pl.BlockSpec((_K, _D), lambda i: (0, 0)),
        ],
        out_specs=[
            pl.BlockSpec((_BLK, _K), lambda i: (i, 0)),
            pl.BlockSpec((_BLK, _D), lambda i: (i, 0)),
            pl.BlockSpec((1, 1), lambda i: (0, 0)),
            pl.BlockSpec((1, 1), lambda i: (0, 0)),
        ],
        out_shape=[
            jax.ShapeDtypeStruct((n, _K), jnp.float32),
            jax.ShapeDtypeStruct((n, _D), jnp.float32),
            jax.ShapeDtypeStruct((1, 1), jnp.float32),
            jax.ShapeDtypeStruct((1, 1), jnp.float32),
        ],
        scratch_shapes=[
            pltpu.SMEM((1, 1), jnp.float32),
            pltpu.VMEM((1, _K), jnp.float32),
            pltpu.VMEM((1, _K), jnp.float32),
        ],
    )(flat, W)
    return (loss[0, 0], q.reshape(input_shape), ppl[0, 0], enc)
